# baseline (speedup 1.0000x reference)
import functools

import jax
import jax.numpy as jnp
from jax import lax
from jax.experimental import pallas as pl
from jax.experimental.pallas import tpu as pltpu
from jax.experimental.pallas import tpu_sc as plsc

N_NODES = 10000
N_EDGES = 320000
NP = 10240
NC, NS = 2, 16
NW = NC * NS
C = 128
KH = -(-N_EDGES // (NS * C))
KD = -(-N_EDGES // (NW * C))
ROWS_PER_TILE = NP // NS

_mesh = plsc.VectorSubcoreMesh(core_axis_name="c", subcore_axis_name="s")


@functools.partial(
    pl.kernel,
    out_type=jax.ShapeDtypeStruct((NC, NP), jnp.float32),
    mesh=_mesh,
    scratch_types=[
        pltpu.VMEM((KD, C), jnp.int32),
        pltpu.VMEM((C,), jnp.float32),
        pltpu.VMEM_SHARED((NP,), jnp.float32),
    ],
)
def _degree_kernel(dst_hbm, ones_hbm, zeros_hbm, out_hbm, idx_v, ones_v,
                   hist_sh):
    c = lax.axis_index("c")
    s = lax.axis_index("s")
    wid = s * NC + c
    base = s * ROWS_PER_TILE
    pltpu.sync_copy(zeros_hbm, hist_sh.at[pl.ds(base, ROWS_PER_TILE)])
    pltpu.sync_copy(ones_hbm, ones_v)
    pltpu.sync_copy(dst_hbm.at[wid], idx_v)
    plsc.subcore_barrier()

    def body(j, carry):
        pltpu.sync_copy(ones_v, hist_sh.at[idx_v.at[j]], add=True)
        return carry

    lax.fori_loop(0, KD, body, 0)
    plsc.subcore_barrier()
    pltpu.sync_copy(hist_sh.at[pl.ds(base, ROWS_PER_TILE)],
                    out_hbm.at[c, pl.ds(base, ROWS_PER_TILE)])


DEPTH = 3
NRING = 2 * DEPTH


def _make_agg_kernel(HD):
    @functools.partial(
        pl.kernel,
        out_type=jax.ShapeDtypeStruct((NC, NP, HD), jnp.float32),
        mesh=_mesh,
        compiler_params=pltpu.CompilerParams(use_tc_tiling_on_sc=False),
        scratch_types=[
            pltpu.VMEM((KH + DEPTH, C), jnp.int32),
            pltpu.VMEM((KH, C), jnp.int32),
            pltpu.VMEM((NRING, C, HD), jnp.float32),
            pltpu.VMEM_SHARED((NP, HD), jnp.float32),
            pltpu.SemaphoreType.DMA((DEPTH,)),
            pltpu.SemaphoreType.DMA((DEPTH,)),
        ],
    )
    def agg_kernel(src_hbm, dst_hbm, g_hbm, zeros_hbm, out_hbm,
                   idx_sv, idx_dv, ring, agg_sh, sem_g, sem_s):
        c = lax.axis_index("c")
        s = lax.axis_index("s")
        base = s * ROWS_PER_TILE
        gv = g_hbm.at[c]
        pltpu.sync_copy(zeros_hbm, agg_sh.at[pl.ds(base, ROWS_PER_TILE)])
        pltpu.sync_copy(src_hbm.at[s], idx_sv)
        pltpu.sync_copy(dst_hbm.at[s], idx_dv)
        plsc.subcore_barrier()

        def gather(j):
            pltpu.async_copy(gv.at[idx_sv.at[j]], ring.at[lax.rem(j, NRING)],
                             sem_g.at[lax.rem(j, DEPTH)])

        def gather_wait(j):
            pltpu.make_async_copy(gv.at[idx_sv.at[j]],
                                  ring.at[lax.rem(j, NRING)],
                                  sem_g.at[lax.rem(j, DEPTH)]).wait()

        def scatter(j):
            pltpu.async_copy(ring.at[lax.rem(j, NRING)],
                             agg_sh.at[idx_dv.at[j]],
                             sem_s.at[lax.rem(j, DEPTH)], add=True)

        def scatter_wait(j):
            pltpu.make_async_copy(ring.at[lax.rem(j, NRING)],
                                  agg_sh.at[idx_dv.at[j]],
                                  sem_s.at[lax.rem(j, DEPTH)]).wait()

        def prologue(i, carry):
            gather(i)
            return carry

        lax.fori_loop(0, DEPTH, prologue, 0)

        def body(j, carry):
            gather_wait(j)

            @pl.when(j >= DEPTH)
            def _():
                scatter_wait(j - DEPTH)

            scatter(j)
            gather(j + DEPTH)
            return carry

        lax.fori_loop(0, KH, body, 0)

        def drain(j, carry):
            gather_wait(j + DEPTH)
            scatter_wait(j)
            return carry

        lax.fori_loop(KH - DEPTH, KH, drain, 0)
        plsc.subcore_barrier()
        pltpu.sync_copy(agg_sh.at[pl.ds(base, ROWS_PER_TILE)],
                        out_hbm.at[c, pl.ds(base, ROWS_PER_TILE)])

    return agg_kernel


_agg_l1 = _make_agg_kernel(64)
_agg_l2 = _make_agg_kernel(32)


_BLK = 1024


def _tc1_body(h0_ref, h1_ref, x_ref, w_ref, dinv_ref, g_ref):
    deg = h0_ref[...] + h1_ref[...] + 1.0
    dinv = lax.rsqrt(deg)
    dinv_ref[...] = dinv
    z = jnp.dot(x_ref[...], w_ref[...], preferred_element_type=jnp.float32)
    g = z * dinv
    g_ref[0] = g[:, :64]
    g_ref[1] = g[:, 64:]


def _tc2_body(a_ref, g_ref, dinv_ref, b_ref, w_ref, g2_ref):
    dinv = dinv_ref[...]
    h = dinv * (a_ref[...] + g_ref[...]) + b_ref[...]
    h = jnp.maximum(h, 0.0)
    h = jnp.concatenate([h[0], h[1]], axis=1)
    g2 = dinv * jnp.dot(h, w_ref[...], preferred_element_type=jnp.float32)
    g2_ref[0] = g2[:, :32]
    g2_ref[1] = g2[:, 32:]


def _tc3_body(a_ref, g_ref, dinv_ref, b_ref, out_ref):
    o = dinv_ref[...] * (a_ref[...] + g_ref[...]) + b_ref[...]
    out_ref[...] = jnp.concatenate([o[0], o[1]], axis=1)


def _row_spec(d):
    return pl.BlockSpec((_BLK, d), lambda i: (i, 0))


def _half_spec(d):
    return pl.BlockSpec((NC, _BLK, d), lambda i: (0, i, 0))


def _full_spec(shape):
    return pl.BlockSpec(shape, lambda i: tuple(0 for _ in shape))


def _tc1(h0, h1, x, w):
    return pl.pallas_call(
        _tc1_body,
        grid=(NP // _BLK,),
        in_specs=[_row_spec(1), _row_spec(1), _row_spec(128),
                  _full_spec((128, 128))],
        out_specs=[_row_spec(1), _half_spec(64)],
        out_shape=[jax.ShapeDtypeStruct((NP, 1), jnp.float32),
                   jax.ShapeDtypeStruct((NC, NP, 64), jnp.float32)],
    )(h0, h1, x, w)


def _tc2(a, g, dinv, b, w):
    return pl.pallas_call(
        _tc2_body,
        grid=(NP // _BLK,),
        in_specs=[_half_spec(64), _half_spec(64), _row_spec(1),
                  _full_spec((NC, 1, 64)), _full_spec((128, 64))],
        out_specs=_half_spec(32),
        out_shape=jax.ShapeDtypeStruct((NC, NP, 32), jnp.float32),
    )(a, g, dinv, b, w)


def _tc3(a, g, dinv, b):
    return pl.pallas_call(
        _tc3_body,
        grid=(NP // _BLK,),
        in_specs=[_half_spec(32), _half_spec(32), _row_spec(1),
                  _full_spec((NC, 1, 32))],
        out_specs=_row_spec(64),
        out_shape=jax.ShapeDtypeStruct((NP, 64), jnp.float32),
    )(a, g, dinv, b)


def kernel(x, edge_index, W1, b1, W2, b2):
    src = edge_index[0].astype(jnp.int32)
    dst = edge_index[1].astype(jnp.int32)

    n_pad = NS * KH * C - N_EDGES
    pad_dst = N_NODES + (jnp.arange(n_pad, dtype=jnp.int32) % (NP - N_NODES))
    src_a = jnp.concatenate([src, jnp.zeros((n_pad,), jnp.int32)])
    dst_a = jnp.concatenate([dst, pad_dst])
    src_a = jnp.pad(src_a.reshape(NS, KH, C), ((0, 0), (0, DEPTH), (0, 0)))
    dst_a = dst_a.reshape(NS, KH, C)

    n_pad_d = NW * KD * C - N_EDGES
    pad_dst_d = N_NODES + (jnp.arange(n_pad_d, dtype=jnp.int32)
                           % (NP - N_NODES))
    dst_d = jnp.concatenate([dst, pad_dst_d]).reshape(NW, KD, C)

    xp = jnp.pad(x, ((0, NP - N_NODES), (0, 0)))
    ones_c = jnp.ones((C,), jnp.float32)
    zeros_1d = jnp.zeros((ROWS_PER_TILE,), jnp.float32)
    zeros_64 = jnp.zeros((ROWS_PER_TILE, 64), jnp.float32)
    zeros_32 = jnp.zeros((ROWS_PER_TILE, 32), jnp.float32)

    hist = _degree_kernel(dst_d, ones_c, zeros_1d)
    h0 = hist[0].reshape(NP, 1)
    h1 = hist[1].reshape(NP, 1)

    dinv, g1 = _tc1(h0, h1, xp, W1)

    agg1 = _agg_l1(src_a, dst_a, g1, zeros_64)
    g2 = _tc2(agg1, g1, dinv, b1.reshape(NC, 1, 64), W2)

    agg2 = _agg_l2(src_a, dst_a, g2, zeros_32)
    out = _tc3(agg2, g2, dinv, b2.reshape(NC, 1, 32))
    return out[:N_NODES]


# speedup vs baseline: 1.0830x; 1.0830x over previous
import functools

import jax
import jax.numpy as jnp
from jax import lax
from jax.experimental import pallas as pl
from jax.experimental.pallas import tpu as pltpu
from jax.experimental.pallas import tpu_sc as plsc

N_NODES = 10000
N_EDGES = 320000
NP = 10240
NC, NS = 2, 16
NW = NC * NS
C = 128
KH = -(-N_EDGES // (NS * C))
KD = -(-N_EDGES // (NW * C))
ROWS_PER_TILE = NP // NS

_mesh = plsc.VectorSubcoreMesh(core_axis_name="c", subcore_axis_name="s")


@functools.partial(
    pl.kernel,
    out_type=jax.ShapeDtypeStruct((NC, NP), jnp.float32),
    mesh=_mesh,
    scratch_types=[
        pltpu.VMEM((KD, C), jnp.int32),
        pltpu.VMEM((C,), jnp.float32),
        pltpu.VMEM_SHARED((NP,), jnp.float32),
    ],
)
def _degree_kernel(dst_hbm, ones_hbm, zeros_hbm, out_hbm, idx_v, ones_v,
                   hist_sh):
    c = lax.axis_index("c")
    s = lax.axis_index("s")
    wid = s * NC + c
    base = s * ROWS_PER_TILE
    pltpu.sync_copy(zeros_hbm, hist_sh.at[pl.ds(base, ROWS_PER_TILE)])
    pltpu.sync_copy(ones_hbm, ones_v)
    pltpu.sync_copy(dst_hbm.at[wid], idx_v)
    plsc.subcore_barrier()

    def body(j, carry):
        pltpu.sync_copy(ones_v, hist_sh.at[idx_v.at[j]], add=True)
        return carry

    lax.fori_loop(0, KD, body, 0)
    plsc.subcore_barrier()
    pltpu.sync_copy(hist_sh.at[pl.ds(base, ROWS_PER_TILE)],
                    out_hbm.at[c, pl.ds(base, ROWS_PER_TILE)])


def _make_agg_kernel(HD):
    @functools.partial(
        pl.kernel,
        out_type=jax.ShapeDtypeStruct((NC, NP, HD), jnp.float32),
        mesh=_mesh,
        compiler_params=pltpu.CompilerParams(use_tc_tiling_on_sc=False),
        scratch_types=[
            pltpu.VMEM((KH + 2, C), jnp.int32),
            pltpu.VMEM((KH, C), jnp.int32),
            pltpu.VMEM((4, C, HD), jnp.float32),
            pltpu.VMEM_SHARED((NP, HD), jnp.float32),
            pltpu.SemaphoreType.DMA,
            pltpu.SemaphoreType.DMA,
            pltpu.SemaphoreType.DMA,
            pltpu.SemaphoreType.DMA,
        ],
    )
    def agg_kernel(src_hbm, dst_hbm, g_hbm, out_hbm,
                   idx_sv, idx_dv, ring, agg_sh, sem_g0, sem_g1, sem_s0,
                   sem_s1):
        c = lax.axis_index("c")
        s = lax.axis_index("s")
        base = s * ROWS_PER_TILE
        gv = g_hbm.at[c]
        pltpu.sync_copy(gv.at[pl.ds(base, ROWS_PER_TILE)],
                        agg_sh.at[pl.ds(base, ROWS_PER_TILE)])
        pltpu.sync_copy(src_hbm.at[s], idx_sv)
        pltpu.sync_copy(dst_hbm.at[s], idx_dv)
        plsc.subcore_barrier()

        def gather(j, sem):
            pltpu.async_copy(gv.at[idx_sv.at[j]], ring.at[lax.rem(j, 4)], sem)

        def gather_wait(j, sem):
            pltpu.make_async_copy(gv.at[idx_sv.at[j]],
                                  ring.at[lax.rem(j, 4)], sem).wait()

        def scatter(j, sem):
            pltpu.async_copy(ring.at[lax.rem(j, 4)], agg_sh.at[idx_dv.at[j]],
                             sem, add=True)

        def scatter_wait(j, sem):
            pltpu.make_async_copy(ring.at[lax.rem(j, 4)],
                                  agg_sh.at[idx_dv.at[j]], sem).wait()

        gather(0, sem_g0)
        gather(1, sem_g1)

        def body2(j, sg, ss):
            gather_wait(j, sg)

            @pl.when(j >= 2)
            def _():
                scatter_wait(j - 2, ss)

            scatter(j, ss)
            gather(j + 2, sg)

        def body(i, carry):
            j0 = 2 * i
            body2(j0, sem_g0, sem_s0)
            body2(j0 + 1, sem_g1, sem_s1)
            return carry

        lax.fori_loop(0, KH // 2, body, 0)
        if KH % 2:
            body2(KH - 1, sem_g0, sem_s0)
            gather_wait(KH, sem_g1)
            gather_wait(KH + 1, sem_g0)
            scatter_wait(KH - 2, sem_s1)
            scatter_wait(KH - 1, sem_s0)
        else:
            gather_wait(KH, sem_g0)
            gather_wait(KH + 1, sem_g1)
            scatter_wait(KH - 2, sem_s0)
            scatter_wait(KH - 1, sem_s1)
        plsc.subcore_barrier()
        pltpu.sync_copy(agg_sh.at[pl.ds(base, ROWS_PER_TILE)],
                        out_hbm.at[c, pl.ds(base, ROWS_PER_TILE)])

    return agg_kernel


_agg_l1 = _make_agg_kernel(64)
_agg_l2 = _make_agg_kernel(32)


_BLK = 1024


def _tc1_body(h_ref, x_ref, w_ref, dinv_ref, g_ref):
    deg = h_ref[0] + h_ref[1] + 1.0
    dinv = lax.rsqrt(deg)
    dinv_ref[...] = dinv
    z = jnp.dot(x_ref[...], w_ref[...], preferred_element_type=jnp.float32)
    g = z * dinv
    g_ref[0] = g[:, :64]
    g_ref[1] = g[:, 64:]


def _tc2_body(a_ref, dinv_ref, b_ref, w_ref, g2_ref):
    dinv = dinv_ref[...]
    h = dinv * a_ref[...] + b_ref[...]
    h = jnp.maximum(h, 0.0)
    h = jnp.concatenate([h[0], h[1]], axis=1)
    g2 = dinv * jnp.dot(h, w_ref[...], preferred_element_type=jnp.float32)
    g2_ref[0] = g2[:, :32]
    g2_ref[1] = g2[:, 32:]


def _tc3_body(a_ref, dinv_ref, b_ref, out_ref):
    o = dinv_ref[...] * a_ref[...] + b_ref[...]
    out_ref[...] = jnp.concatenate([o[0], o[1]], axis=1)


def _row_spec(d):
    return pl.BlockSpec((_BLK, d), lambda i: (i, 0))


def _half_spec(d):
    return pl.BlockSpec((NC, _BLK, d), lambda i: (0, i, 0))


def _full_spec(shape):
    return pl.BlockSpec(shape, lambda i: tuple(0 for _ in shape))


def _tc1(h, x, w):
    return pl.pallas_call(
        _tc1_body,
        grid=(NP // _BLK,),
        in_specs=[_half_spec(1), _row_spec(128), _full_spec((128, 128))],
        out_specs=[_row_spec(1), _half_spec(64)],
        out_shape=[jax.ShapeDtypeStruct((NP, 1), jnp.float32),
                   jax.ShapeDtypeStruct((NC, NP, 64), jnp.float32)],
    )(h, x, w)


def _tc2(a, dinv, b, w):
    return pl.pallas_call(
        _tc2_body,
        grid=(NP // _BLK,),
        in_specs=[_half_spec(64), _row_spec(1),
                  _full_spec((NC, 1, 64)), _full_spec((128, 64))],
        out_specs=_half_spec(32),
        out_shape=jax.ShapeDtypeStruct((NC, NP, 32), jnp.float32),
    )(a, dinv, b, w)


def _tc3(a, dinv, b):
    return pl.pallas_call(
        _tc3_body,
        grid=(NP // _BLK,),
        in_specs=[_half_spec(32), _row_spec(1), _full_spec((NC, 1, 32))],
        out_specs=_row_spec(64),
        out_shape=jax.ShapeDtypeStruct((NP, 64), jnp.float32),
    )(a, dinv, b)


def kernel(x, edge_index, W1, b1, W2, b2):
    src = edge_index[0].astype(jnp.int32)
    dst = edge_index[1].astype(jnp.int32)

    n_pad = NS * KH * C - N_EDGES
    pad_dst = N_NODES + (jnp.arange(n_pad, dtype=jnp.int32) % (NP - N_NODES))
    src_a = jnp.concatenate([src, jnp.zeros((n_pad,), jnp.int32)])
    dst_a = jnp.concatenate([dst, pad_dst])
    src_a = jnp.pad(src_a.reshape(NS, KH, C), ((0, 0), (0, 2), (0, 0)))
    dst_a = dst_a.reshape(NS, KH, C)

    n_pad_d = NW * KD * C - N_EDGES
    pad_dst_d = N_NODES + (jnp.arange(n_pad_d, dtype=jnp.int32)
                           % (NP - N_NODES))
    dst_d = jnp.concatenate([dst, pad_dst_d]).reshape(NW, KD, C)

    xp = jnp.pad(x, ((0, NP - N_NODES), (0, 0)))
    ones_c = jnp.ones((C,), jnp.float32)
    zeros_1d = jnp.zeros((ROWS_PER_TILE,), jnp.float32)

    hist = _degree_kernel(dst_d, ones_c, zeros_1d)

    dinv, g1 = _tc1(hist.reshape(NC, NP, 1), xp, W1)

    agg1 = _agg_l1(src_a, dst_a, g1)
    g2 = _tc2(agg1, dinv, b1.reshape(NC, 1, 64), W2)

    agg2 = _agg_l2(src_a, dst_a, g2)
    out = _tc3(agg2, dinv, b2.reshape(NC, 1, 32))
    return out[:N_NODES]
